# Initial kernel scaffold; baseline (speedup 1.0000x reference)
import jax
import jax.numpy as jnp
from jax.experimental import pallas as pl


def kernel(node_inputs, edge_inputs, edge_index, Wn1, bn1, We1a, be1a, We1b, be1b, Wo1, bo1, Wn2, bn2, We2a, be2a, We2b, be2b, Wo2, bo2):
    raise NotImplementedError("write your pallas kernel here")


# speedup vs baseline: 1.9643x; 1.9643x over previous
import functools

import jax
import jax.numpy as jnp
from jax import lax
from jax.experimental import pallas as pl
from jax.experimental.pallas import tpu as pltpu
from jax.experimental.pallas import tpu_sc as plsc

N = 10000
E = 320000
D_NODE = 128
D_EDGE = 16
D = 128

CHUNK = 80
T_EDGES = E // 16
TCH = T_EDGES // CHUNK
HALF = N // 2
ACC_ROWS = HALF + 8

_LOG2 = 0.6931471805599453


def _ssp(x):
    return jnp.maximum(x, 0.0) + jnp.log1p(jnp.exp(-jnp.abs(x))) - _LOG2


def _nodeproj_body(x_ref, w_ref, b_ref, o_ref):
    o_ref[...] = jnp.dot(x_ref[...], w_ref[...],
                         preferred_element_type=jnp.float32) + b_ref[...]


def _node_proj(x, w, b2d):
    blk = 1000
    return pl.pallas_call(
        _nodeproj_body,
        grid=(N // blk,),
        in_specs=[
            pl.BlockSpec((blk, D), lambda i: (i, 0)),
            pl.BlockSpec((D, D), lambda i: (0, 0)),
            pl.BlockSpec((1, D), lambda i: (0, 0)),
        ],
        out_specs=pl.BlockSpec((blk, D), lambda i: (i, 0)),
        out_shape=jax.ShapeDtypeStruct((N, D), jnp.float32),
    )(x, w, b2d)


def _edge_body(e_ref, w1a_ref, b1a_ref, w1b_ref, b1b_ref,
               w2a_ref, b2a_ref, w2b_ref, b2b_ref, he1_ref, he2_ref):
    e = e_ref[...]
    h1 = _ssp(jnp.dot(e, w1a_ref[...], preferred_element_type=jnp.float32)
              + b1a_ref[...])
    he1_ref[...] = _ssp(jnp.dot(h1, w1b_ref[...],
                                preferred_element_type=jnp.float32)
                        + b1b_ref[...])
    h2 = _ssp(jnp.dot(e, w2a_ref[...], preferred_element_type=jnp.float32)
              + b2a_ref[...])
    he2_ref[...] = _ssp(jnp.dot(h2, w2b_ref[...],
                                preferred_element_type=jnp.float32)
                        + b2b_ref[...])


def _edge_mlp_dual(e, w1a, b1a, w1b, b1b, w2a, b2a, w2b, b2b):
    blk = 2000
    wspec = pl.BlockSpec((D, D), lambda i: (0, 0))
    bspec = pl.BlockSpec((1, D), lambda i: (0, 0))
    return pl.pallas_call(
        _edge_body,
        grid=(E // blk,),
        in_specs=[
            pl.BlockSpec((blk, D_EDGE), lambda i: (i, 0)),
            pl.BlockSpec((D_EDGE, D), lambda i: (0, 0)), bspec,
            wspec, bspec,
            pl.BlockSpec((D_EDGE, D), lambda i: (0, 0)), bspec,
            wspec, bspec,
        ],
        out_specs=[
            pl.BlockSpec((blk, D), lambda i: (i, 0)),
            pl.BlockSpec((blk, D), lambda i: (i, 0)),
        ],
        out_shape=[
            jax.ShapeDtypeStruct((E, D), jnp.float32),
            jax.ShapeDtypeStruct((E, D), jnp.float32),
        ],
    )(e, w1a, b1a, w1b, b1b, w2a, b2a, w2b, b2b)


def _mid_body(p_ref, wo_ref, bo_ref, wn_ref, bn_ref, o_ref):
    agg = p_ref[0]
    t = jnp.tanh(_ssp(jnp.dot(agg, wo_ref[...],
                              preferred_element_type=jnp.float32)
                      + bo_ref[...]))
    o_ref[...] = jnp.dot(t, wn_ref[...],
                         preferred_element_type=jnp.float32) + bn_ref[...]


def _mid_proj(p, wo, bo2d, wn, bn2d):
    blk = 1000
    wspec = pl.BlockSpec((D, D), lambda i: (0, 0))
    bspec = pl.BlockSpec((1, D), lambda i: (0, 0))
    return pl.pallas_call(
        _mid_body,
        grid=(N // blk,),
        in_specs=[
            pl.BlockSpec((1, blk, D), lambda i: (i // 5, i % 5, 0)),
            wspec, bspec, wspec, bspec,
        ],
        out_specs=pl.BlockSpec((blk, D), lambda i: (i, 0)),
        out_shape=jax.ShapeDtypeStruct((N, D), jnp.float32),
    )(p, wo, bo2d, wn, bn2d)


def _fin_body(p_ref, wo_ref, bo_ref, o_ref):
    agg = p_ref[0]
    o_ref[...] = jnp.tanh(_ssp(jnp.dot(agg, wo_ref[...],
                                       preferred_element_type=jnp.float32)
                               + bo_ref[...]))


def _fin_proj(p, wo, bo2d):
    blk = 1000
    return pl.pallas_call(
        _fin_body,
        grid=(N // blk,),
        in_specs=[
            pl.BlockSpec((1, blk, D), lambda i: (i // 5, i % 5, 0)),
            pl.BlockSpec((D, D), lambda i: (0, 0)),
            pl.BlockSpec((1, D), lambda i: (0, 0)),
        ],
        out_specs=pl.BlockSpec((blk, D), lambda i: (i, 0)),
        out_shape=jax.ShapeDtypeStruct((N, D), jnp.float32),
    )(p, wo, bo2d)


def _sc_body(hv_hbm, he_hbm, src_hbm, dst_hbm, out_hbm,
             src_v, dst_v, tidx, gbuf, hbuf, acc, sem1, sem2):
    c = lax.axis_index("c")
    s = lax.axis_index("s")
    base = c * HALF

    pltpu.sync_copy(src_hbm.at[s], src_v)
    pltpu.sync_copy(dst_hbm.at[s], dst_v)

    def zbody(r, carry):
        for t in range(8):
            gbuf[r, pl.ds(t * 16, 16)] = jnp.zeros((16,), jnp.float32)
        return carry
    lax.fori_loop(0, CHUNK, zbody, 0)
    for g in range(63):
        sz = CHUNK if g < 62 else ACC_ROWS - 62 * CHUNK

        @pl.when(g % 16 == s)
        def _():
            pltpu.sync_copy(gbuf.at[pl.ds(0, sz)],
                            acc.at[pl.ds(g * CHUNK, sz)])
    plsc.subcore_barrier()

    def chunk_body(j, carry):
        cp1 = pltpu.async_copy(hv_hbm.at[src_v.at[j]], gbuf, sem1)
        cp2 = pltpu.async_copy(
            he_hbm.at[pl.ds(s * T_EDGES + j * CHUNK, CHUNK)], hbuf, sem2)
        for k in range(5):
            sl = pl.ds(k * 16, 16)
            d = dst_v[j, sl] - base
            oob = (d < 0) | (d >= HALF)
            tidx[sl] = jnp.where(oob, HALF + (d & 7), d)
        cp1.wait()
        cp2.wait()

        def mbody(r, inner):
            for t in range(8):
                sl = pl.ds(t * 16, 16)
                gbuf[r, sl] = gbuf[r, sl] * hbuf[r, sl]
            return inner
        lax.fori_loop(0, CHUNK, mbody, 0)
        pltpu.sync_copy(gbuf, acc.at[tidx], add=True)
        return carry
    lax.fori_loop(0, TCH, chunk_body, 0)

    plsc.subcore_barrier()
    for g in range(63):
        sz = CHUNK if g < 62 else HALF - 62 * CHUNK

        @pl.when(g % 16 == s)
        def _():
            pltpu.sync_copy(acc.at[pl.ds(g * CHUNK, sz)],
                            gbuf.at[pl.ds(0, sz)])
            pltpu.sync_copy(gbuf.at[pl.ds(0, sz)],
                            out_hbm.at[c, pl.ds(g * CHUNK, sz)])


@functools.cache
def _make_sc_gms():
    return pl.kernel(
        _sc_body,
        out_type=jax.ShapeDtypeStruct((2, HALF, D), jnp.float32),
        mesh=plsc.VectorSubcoreMesh(core_axis_name="c", subcore_axis_name="s"),
        scratch_types=[
            pltpu.VMEM((TCH, CHUNK), jnp.int32),
            pltpu.VMEM((TCH, CHUNK), jnp.int32),
            pltpu.VMEM((CHUNK,), jnp.int32),
            pltpu.VMEM((CHUNK, D), jnp.float32),
            pltpu.VMEM((CHUNK, D), jnp.float32),
            pltpu.VMEM_SHARED((ACC_ROWS, D), jnp.float32),
            pltpu.SemaphoreType.DMA,
            pltpu.SemaphoreType.DMA,
        ],
    )


def _sc_gms(hv, he, src, dst):
    return _make_sc_gms()(hv, he, src, dst)


def kernel(node_inputs, edge_inputs, edge_index,
           Wn1, bn1, We1a, be1a, We1b, be1b, Wo1, bo1,
           Wn2, bn2, We2a, be2a, We2b, be2b, Wo2, bo2):
    src = edge_index[0].reshape(16, TCH, CHUNK)
    dst = edge_index[1].reshape(16, TCH, CHUNK)

    bn1_2 = bn1.reshape(1, D)
    be1a_2 = be1a.reshape(1, D)
    be1b_2 = be1b.reshape(1, D)
    bo1_2 = bo1.reshape(1, D)
    bn2_2 = bn2.reshape(1, D)
    be2a_2 = be2a.reshape(1, D)
    be2b_2 = be2b.reshape(1, D)
    bo2_2 = bo2.reshape(1, D)

    he1, he2 = _edge_mlp_dual(edge_inputs, We1a, be1a_2, We1b, be1b_2,
                              We2a, be2a_2, We2b, be2b_2)
    hv1 = _node_proj(node_inputs, Wn1, bn1_2)
    p1 = _sc_gms(hv1, he1, src, dst)
    hv2 = _mid_proj(p1, Wo1, bo1_2, Wn2, bn2_2)
    p2 = _sc_gms(hv2, he2, src, dst)
    return _fin_proj(p2, Wo2, bo2_2)


# speedup vs baseline: 2.8427x; 1.4472x over previous
import functools

import jax
import jax.numpy as jnp
from jax import lax
from jax.experimental import pallas as pl
from jax.experimental.pallas import tpu as pltpu
from jax.experimental.pallas import tpu_sc as plsc

N = 10000
E = 320000
D_NODE = 128
D_EDGE = 16
D = 128

CHUNK = 80
T_EDGES = E // 16
TCH = T_EDGES // CHUNK
SUP = 25
NSUP = TCH // SUP
CAP = T_EDGES + 2 * CHUNK
HALF = N // 2
ACC_ROWS = HALF + 8

_LOG2 = 0.6931471805599453


def _ssp(x):
    return jnp.maximum(x, 0.0) + jnp.log1p(jnp.exp(-jnp.abs(x))) - _LOG2


def _nodeproj_body(x_ref, w_ref, b_ref, o_ref):
    o_ref[...] = jnp.dot(x_ref[...], w_ref[...],
                         preferred_element_type=jnp.float32) + b_ref[...]


def _node_proj(x, w, b2d):
    blk = 1000
    return pl.pallas_call(
        _nodeproj_body,
        grid=(N // blk,),
        in_specs=[
            pl.BlockSpec((blk, D), lambda i: (i, 0)),
            pl.BlockSpec((D, D), lambda i: (0, 0)),
            pl.BlockSpec((1, D), lambda i: (0, 0)),
        ],
        out_specs=pl.BlockSpec((blk, D), lambda i: (i, 0)),
        out_shape=jax.ShapeDtypeStruct((N, D), jnp.float32),
    )(x, w, b2d)


def _edge_body(e_ref, w1a_ref, b1a_ref, w1b_ref, b1b_ref,
               w2a_ref, b2a_ref, w2b_ref, b2b_ref, he1_ref, he2_ref):
    e = e_ref[...]
    h1 = _ssp(jnp.dot(e, w1a_ref[...], preferred_element_type=jnp.float32)
              + b1a_ref[...])
    he1_ref[...] = _ssp(jnp.dot(h1, w1b_ref[...],
                                preferred_element_type=jnp.float32)
                        + b1b_ref[...])
    h2 = _ssp(jnp.dot(e, w2a_ref[...], preferred_element_type=jnp.float32)
              + b2a_ref[...])
    he2_ref[...] = _ssp(jnp.dot(h2, w2b_ref[...],
                                preferred_element_type=jnp.float32)
                        + b2b_ref[...])


def _edge_mlp_dual(e, w1a, b1a, w1b, b1b, w2a, b2a, w2b, b2b):
    blk = 2000
    wspec = pl.BlockSpec((D, D), lambda i: (0, 0))
    bspec = pl.BlockSpec((1, D), lambda i: (0, 0))
    return pl.pallas_call(
        _edge_body,
        grid=(E // blk,),
        in_specs=[
            pl.BlockSpec((blk, D_EDGE), lambda i: (i, 0)),
            pl.BlockSpec((D_EDGE, D), lambda i: (0, 0)), bspec,
            wspec, bspec,
            pl.BlockSpec((D_EDGE, D), lambda i: (0, 0)), bspec,
            wspec, bspec,
        ],
        out_specs=[
            pl.BlockSpec((blk, D), lambda i: (i, 0)),
            pl.BlockSpec((blk, D), lambda i: (i, 0)),
        ],
        out_shape=[
            jax.ShapeDtypeStruct((E, D), jnp.float32),
            jax.ShapeDtypeStruct((E, D), jnp.float32),
        ],
    )(e, w1a, b1a, w1b, b1b, w2a, b2a, w2b, b2b)


def _mid_body(p_ref, wo_ref, bo_ref, wn_ref, bn_ref, o_ref):
    agg = p_ref[0]
    t = jnp.tanh(_ssp(jnp.dot(agg, wo_ref[...],
                              preferred_element_type=jnp.float32)
                      + bo_ref[...]))
    o_ref[...] = jnp.dot(t, wn_ref[...],
                         preferred_element_type=jnp.float32) + bn_ref[...]


def _mid_proj(p, wo, bo2d, wn, bn2d):
    blk = 1000
    wspec = pl.BlockSpec((D, D), lambda i: (0, 0))
    bspec = pl.BlockSpec((1, D), lambda i: (0, 0))
    return pl.pallas_call(
        _mid_body,
        grid=(N // blk,),
        in_specs=[
            pl.BlockSpec((1, blk, D), lambda i: (i // 5, i % 5, 0)),
            wspec, bspec, wspec, bspec,
        ],
        out_specs=pl.BlockSpec((blk, D), lambda i: (i, 0)),
        out_shape=jax.ShapeDtypeStruct((N, D), jnp.float32),
    )(p, wo, bo2d, wn, bn2d)


def _fin_body(p_ref, wo_ref, bo_ref, o_ref):
    agg = p_ref[0]
    o_ref[...] = jnp.tanh(_ssp(jnp.dot(agg, wo_ref[...],
                                       preferred_element_type=jnp.float32)
                               + bo_ref[...]))


def _fin_proj(p, wo, bo2d):
    blk = 1000
    return pl.pallas_call(
        _fin_body,
        grid=(N // blk,),
        in_specs=[
            pl.BlockSpec((1, blk, D), lambda i: (i // 5, i % 5, 0)),
            pl.BlockSpec((D, D), lambda i: (0, 0)),
            pl.BlockSpec((1, D), lambda i: (0, 0)),
        ],
        out_specs=pl.BlockSpec((blk, D), lambda i: (i, 0)),
        out_shape=jax.ShapeDtypeStruct((N, D), jnp.float32),
    )(p, wo, bo2d)


def _sc_body(hv_hbm, he_hbm, src_hbm, dst_hbm, out_hbm,
             src_v, dst_v, tidx, gbuf0, gbuf1, hbuf0, hbuf1, acc,
             gsem0, gsem1, hsem0, hsem1):
    c = lax.axis_index("c")
    s = lax.axis_index("s")
    base = c * HALF
    gbufs = (gbuf0, gbuf1)
    hbufs = (hbuf0, hbuf1)
    gsems = (gsem0, gsem1)
    hsems = (hsem0, hsem1)

    pltpu.sync_copy(src_hbm.at[pl.ds(s * T_EDGES, T_EDGES)], src_v)
    pltpu.sync_copy(dst_hbm.at[pl.ds(s * T_EDGES, T_EDGES)], dst_v)

    def zbody(r, carry):
        for t in range(8):
            gbuf0[r, pl.ds(t * 16, 16)] = jnp.zeros((16,), jnp.float32)
        return carry
    lax.fori_loop(0, CHUNK, zbody, 0)
    for g in range(63):
        sz = CHUNK if g < 62 else ACC_ROWS - 62 * CHUNK

        @pl.when(g % 16 == s)
        def _():
            pltpu.sync_copy(gbuf0.at[pl.ds(0, sz)],
                            acc.at[pl.ds(g * CHUNK, sz)])
    plsc.subcore_barrier()

    def fire(chunk, b):
        pltpu.async_copy(hv_hbm.at[src_v.at[pl.ds(chunk * CHUNK, CHUNK)]],
                         gbufs[b], gsems[b])
        pltpu.async_copy(
            he_hbm.at[pl.ds(s * T_EDGES + chunk * CHUNK, CHUNK)],
            hbufs[b], hsems[b])

    fire(0, 0)
    fire(1, 1)

    def pair_body(j2, carry):
        for b in (0, 1):
            chunk = j2 * 2 + b
            pltpu.make_async_copy(hv_hbm.at[pl.ds(0, CHUNK)],
                                  gbufs[b], gsems[b]).wait()
            pltpu.make_async_copy(he_hbm.at[pl.ds(0, CHUNK)],
                                  hbufs[b], hsems[b]).wait()
            for k in range(5):
                sl = pl.ds(k * 16, 16)
                d = dst_v[pl.ds(chunk * CHUNK + k * 16, 16)] - base
                oob = (d < 0) | (d >= HALF)
                tidx[sl] = jnp.where(oob, HALF + (d & 7), d)

            def mbody(r, inner):
                for t in range(8):
                    sl = pl.ds(t * 16, 16)
                    gbufs[b][r, sl] = gbufs[b][r, sl] * hbufs[b][r, sl]
                return inner
            lax.fori_loop(0, CHUNK, mbody, 0)
            pltpu.sync_copy(gbufs[b], acc.at[tidx], add=True)

            @pl.when(chunk + 2 < TCH)
            def _():
                fire(chunk + 2, b)
        return carry
    lax.fori_loop(0, TCH // 2, pair_body, 0)

    plsc.subcore_barrier()
    for g in range(63):
        sz = CHUNK if g < 62 else HALF - 62 * CHUNK

        @pl.when(g % 16 == s)
        def _():
            pltpu.sync_copy(acc.at[pl.ds(g * CHUNK, sz)],
                            gbuf0.at[pl.ds(0, sz)])
            pltpu.sync_copy(gbuf0.at[pl.ds(0, sz)],
                            out_hbm.at[c, pl.ds(g * CHUNK, sz)])


@functools.cache
def _make_sc_gms():
    return pl.kernel(
        _sc_body,
        out_type=jax.ShapeDtypeStruct((2, HALF, D), jnp.float32),
        mesh=plsc.VectorSubcoreMesh(core_axis_name="c", subcore_axis_name="s"),
        scratch_types=[
            pltpu.VMEM((T_EDGES,), jnp.int32),
            pltpu.VMEM((T_EDGES,), jnp.int32),
            pltpu.VMEM((CHUNK,), jnp.int32),
            pltpu.VMEM((CHUNK, D), jnp.float32),
            pltpu.VMEM((CHUNK, D), jnp.float32),
            pltpu.VMEM((CHUNK, D), jnp.float32),
            pltpu.VMEM((CHUNK, D), jnp.float32),
            pltpu.VMEM_SHARED((ACC_ROWS, D), jnp.float32),
            pltpu.SemaphoreType.DMA,
            pltpu.SemaphoreType.DMA,
            pltpu.SemaphoreType.DMA,
            pltpu.SemaphoreType.DMA,
        ],
    )


def _sc_gms(hv, he, src, dst):
    return _make_sc_gms()(hv, he, src, dst)


def kernel(node_inputs, edge_inputs, edge_index,
           Wn1, bn1, We1a, be1a, We1b, be1b, Wo1, bo1,
           Wn2, bn2, We2a, be2a, We2b, be2b, Wo2, bo2):
    src = edge_index[0]
    dst = edge_index[1]

    bn1_2 = bn1.reshape(1, D)
    be1a_2 = be1a.reshape(1, D)
    be1b_2 = be1b.reshape(1, D)
    bo1_2 = bo1.reshape(1, D)
    bn2_2 = bn2.reshape(1, D)
    be2a_2 = be2a.reshape(1, D)
    be2b_2 = be2b.reshape(1, D)
    bo2_2 = bo2.reshape(1, D)

    he1, he2 = _edge_mlp_dual(edge_inputs, We1a, be1a_2, We1b, be1b_2,
                              We2a, be2a_2, We2b, be2b_2)
    hv1 = _node_proj(node_inputs, Wn1, bn1_2)
    p1 = _sc_gms(hv1, he1, src, dst)
    hv2 = _mid_proj(p1, Wo1, bo1_2, Wn2, bn2_2)
    p2 = _sc_gms(hv2, he2, src, dst)
    return _fin_proj(p2, Wo2, bo2_2)


# speedup vs baseline: 3.2089x; 1.1288x over previous
import functools

import jax
import jax.numpy as jnp
from jax import lax
from jax.experimental import pallas as pl
from jax.experimental.pallas import tpu as pltpu
from jax.experimental.pallas import tpu_sc as plsc

N = 10000
E = 320000
D_NODE = 128
D_EDGE = 16
D = 128

CHUNK = 80
T_EDGES = E // 16
TCH = T_EDGES // CHUNK
SUP = 25
NSUP = TCH // SUP
CAP = T_EDGES + 2 * CHUNK
HALF = N // 2
ACC_ROWS = HALF + 8

_LOG2 = 0.6931471805599453


def _ssp(x):
    return jnp.log1p(jnp.exp(x)) - _LOG2


def _nodeproj_body(x_ref, w_ref, b_ref, o_ref):
    o_ref[...] = jnp.dot(x_ref[...], w_ref[...],
                         preferred_element_type=jnp.float32) + b_ref[...]


def _node_proj(x, w, b2d):
    blk = 1000
    return pl.pallas_call(
        _nodeproj_body,
        grid=(N // blk,),
        in_specs=[
            pl.BlockSpec((blk, D), lambda i: (i, 0)),
            pl.BlockSpec((D, D), lambda i: (0, 0)),
            pl.BlockSpec((1, D), lambda i: (0, 0)),
        ],
        out_specs=pl.BlockSpec((blk, D), lambda i: (i, 0)),
        out_shape=jax.ShapeDtypeStruct((N, D), jnp.float32),
    )(x, w, b2d)


def _edge_body(e_ref, wa_ref, ba_ref, wb_ref, bb_ref, he_ref):
    e = e_ref[...]
    h1 = _ssp(jnp.dot(e, wa_ref[...], preferred_element_type=jnp.float32)
              + ba_ref[...])
    he_ref[...] = _ssp(jnp.dot(h1, wb_ref[...],
                               preferred_element_type=jnp.float32)
                       + bb_ref[...])


def _edge_mlp(e, wa, ba, wb, bb):
    blk = 2000
    return pl.pallas_call(
        _edge_body,
        grid=(E // blk,),
        in_specs=[
            pl.BlockSpec((blk, D_EDGE), lambda i: (i, 0)),
            pl.BlockSpec((D_EDGE, D), lambda i: (0, 0)),
            pl.BlockSpec((1, D), lambda i: (0, 0)),
            pl.BlockSpec((D, D), lambda i: (0, 0)),
            pl.BlockSpec((1, D), lambda i: (0, 0)),
        ],
        out_specs=pl.BlockSpec((blk, D), lambda i: (i, 0)),
        out_shape=jax.ShapeDtypeStruct((E, D), jnp.float32),
    )(e, wa, ba, wb, bb)


def _mid_body(p_ref, wo_ref, bo_ref, wn_ref, bn_ref, o_ref):
    agg = p_ref[0]
    t = jnp.tanh(_ssp(jnp.dot(agg, wo_ref[...],
                              preferred_element_type=jnp.float32)
                      + bo_ref[...]))
    o_ref[...] = jnp.dot(t, wn_ref[...],
                         preferred_element_type=jnp.float32) + bn_ref[...]


def _mid_proj(p, wo, bo2d, wn, bn2d):
    blk = 1000
    wspec = pl.BlockSpec((D, D), lambda i: (0, 0))
    bspec = pl.BlockSpec((1, D), lambda i: (0, 0))
    return pl.pallas_call(
        _mid_body,
        grid=(N // blk,),
        in_specs=[
            pl.BlockSpec((1, blk, D), lambda i: (i // 5, i % 5, 0)),
            wspec, bspec, wspec, bspec,
        ],
        out_specs=pl.BlockSpec((blk, D), lambda i: (i, 0)),
        out_shape=jax.ShapeDtypeStruct((N, D), jnp.float32),
    )(p, wo, bo2d, wn, bn2d)


def _fin_body(p_ref, wo_ref, bo_ref, o_ref):
    agg = p_ref[0]
    o_ref[...] = jnp.tanh(_ssp(jnp.dot(agg, wo_ref[...],
                                       preferred_element_type=jnp.float32)
                               + bo_ref[...]))


def _fin_proj(p, wo, bo2d):
    blk = 1000
    return pl.pallas_call(
        _fin_body,
        grid=(N // blk,),
        in_specs=[
            pl.BlockSpec((1, blk, D), lambda i: (i // 5, i % 5, 0)),
            pl.BlockSpec((D, D), lambda i: (0, 0)),
            pl.BlockSpec((1, D), lambda i: (0, 0)),
        ],
        out_specs=pl.BlockSpec((blk, D), lambda i: (i, 0)),
        out_shape=jax.ShapeDtypeStruct((N, D), jnp.float32),
    )(p, wo, bo2d)


def _sc_body(hv_hbm, he_hbm, src_hbm, dst_hbm, out_hbm,
             src_v, dst_v, tidx, gbuf0, gbuf1, hbuf0, hbuf1, acc,
             gsem0, gsem1, hsem0, hsem1):
    c = lax.axis_index("c")
    s = lax.axis_index("s")
    base = c * HALF
    gbufs = (gbuf0, gbuf1)
    hbufs = (hbuf0, hbuf1)
    gsems = (gsem0, gsem1)
    hsems = (hsem0, hsem1)

    pltpu.sync_copy(src_hbm.at[pl.ds(s * T_EDGES, T_EDGES)], src_v)
    pltpu.sync_copy(dst_hbm.at[pl.ds(s * T_EDGES, T_EDGES)], dst_v)

    def zbody(r, carry):
        for t in range(8):
            gbuf0[r, pl.ds(t * 16, 16)] = jnp.zeros((16,), jnp.float32)
        return carry
    lax.fori_loop(0, CHUNK, zbody, 0)
    for g in range(63):
        sz = CHUNK if g < 62 else ACC_ROWS - 62 * CHUNK

        @pl.when(g % 16 == s)
        def _():
            pltpu.sync_copy(gbuf0.at[pl.ds(0, sz)],
                            acc.at[pl.ds(g * CHUNK, sz)])
    plsc.subcore_barrier()

    def fire(chunk, b):
        pltpu.async_copy(hv_hbm.at[src_v.at[pl.ds(chunk * CHUNK, CHUNK)]],
                         gbufs[b], gsems[b])
        pltpu.async_copy(
            he_hbm.at[pl.ds(s * T_EDGES + chunk * CHUNK, CHUNK)],
            hbufs[b], hsems[b])

    fire(0, 0)
    fire(1, 1)

    def pair_body(j2, carry):
        for b in (0, 1):
            chunk = j2 * 2 + b
            pltpu.make_async_copy(hv_hbm.at[pl.ds(0, CHUNK)],
                                  gbufs[b], gsems[b]).wait()
            pltpu.make_async_copy(he_hbm.at[pl.ds(0, CHUNK)],
                                  hbufs[b], hsems[b]).wait()
            for k in range(5):
                sl = pl.ds(k * 16, 16)
                d = dst_v[pl.ds(chunk * CHUNK + k * 16, 16)] - base
                oob = (d < 0) | (d >= HALF)
                tidx[sl] = jnp.where(oob, HALF + (d & 7), d)

            def mbody(r, inner):
                for t in range(8):
                    sl = pl.ds(t * 16, 16)
                    gbufs[b][r, sl] = gbufs[b][r, sl] * hbufs[b][r, sl]
                return inner
            lax.fori_loop(0, CHUNK, mbody, 0)
            pltpu.sync_copy(gbufs[b], acc.at[tidx], add=True)

            @pl.when(chunk + 2 < TCH)
            def _():
                fire(chunk + 2, b)
        return carry
    lax.fori_loop(0, TCH // 2, pair_body, 0)

    plsc.subcore_barrier()
    for g in range(63):
        sz = CHUNK if g < 62 else HALF - 62 * CHUNK

        @pl.when(g % 16 == s)
        def _():
            pltpu.sync_copy(acc.at[pl.ds(g * CHUNK, sz)],
                            gbuf0.at[pl.ds(0, sz)])
            pltpu.sync_copy(gbuf0.at[pl.ds(0, sz)],
                            out_hbm.at[c, pl.ds(g * CHUNK, sz)])


@functools.cache
def _make_sc_gms():
    return pl.kernel(
        _sc_body,
        out_type=jax.ShapeDtypeStruct((2, HALF, D), jnp.float32),
        mesh=plsc.VectorSubcoreMesh(core_axis_name="c", subcore_axis_name="s"),
        scratch_types=[
            pltpu.VMEM((T_EDGES,), jnp.int32),
            pltpu.VMEM((T_EDGES,), jnp.int32),
            pltpu.VMEM((CHUNK,), jnp.int32),
            pltpu.VMEM((CHUNK, D), jnp.float32),
            pltpu.VMEM((CHUNK, D), jnp.float32),
            pltpu.VMEM((CHUNK, D), jnp.float32),
            pltpu.VMEM((CHUNK, D), jnp.float32),
            pltpu.VMEM_SHARED((ACC_ROWS, D), jnp.float32),
            pltpu.SemaphoreType.DMA,
            pltpu.SemaphoreType.DMA,
            pltpu.SemaphoreType.DMA,
            pltpu.SemaphoreType.DMA,
        ],
    )


def _sc_gms(hv, he, src, dst):
    return _make_sc_gms()(hv, he, src, dst)


def kernel(node_inputs, edge_inputs, edge_index,
           Wn1, bn1, We1a, be1a, We1b, be1b, Wo1, bo1,
           Wn2, bn2, We2a, be2a, We2b, be2b, Wo2, bo2):
    src = edge_index[0]
    dst = edge_index[1]

    bn1_2 = bn1.reshape(1, D)
    be1a_2 = be1a.reshape(1, D)
    be1b_2 = be1b.reshape(1, D)
    bo1_2 = bo1.reshape(1, D)
    bn2_2 = bn2.reshape(1, D)
    be2a_2 = be2a.reshape(1, D)
    be2b_2 = be2b.reshape(1, D)
    bo2_2 = bo2.reshape(1, D)

    he1 = _edge_mlp(edge_inputs, We1a, be1a_2, We1b, be1b_2)
    hv1 = _node_proj(node_inputs, Wn1, bn1_2)
    p1 = _sc_gms(hv1, he1, src, dst)
    he2 = _edge_mlp(edge_inputs, We2a, be2a_2, We2b, be2b_2)
    hv2 = _mid_proj(p1, Wo1, bo1_2, Wn2, bn2_2)
    p2 = _sc_gms(hv2, he2, src, dst)
    return _fin_proj(p2, Wo2, bo2_2)
